# Initial kernel scaffold; baseline (speedup 1.0000x reference)
import jax
import jax.numpy as jnp
from jax.experimental import pallas as pl


def kernel(x, edge_index, batch, W0, b0, W1, b1, W2, b2, W3, b3, Wc1, bc1, Wc2, bc2, Wd1, bd1, Wd2, bd2):
    raise NotImplementedError("write your pallas kernel here")


# speedup vs baseline: 4.8664x; 4.8664x over previous
import functools

import jax
import jax.numpy as jnp
from jax import lax
from jax.experimental import pallas as pl
from jax.experimental.pallas import tpu as pltpu
from jax.experimental.pallas import tpu_sc as plsc

_N = 10000
_E = 320000
_B = 64
_K = 64
_NPAD = 10240
_RPT = _NPAD // 16
_VPT = _N // 16
_CH = 128


def _make_segsum(d, nch):
  mesh = plsc.VectorSubcoreMesh(core_axis_name="c", subcore_axis_name="s")

  seg = nch // 2

  @functools.partial(
      pl.kernel,
      out_type=jax.ShapeDtypeStruct((2, _N, d), jnp.float32),
      mesh=mesh,
      scratch_types=[
          pltpu.VMEM((seg, _CH), jnp.int32),
          pltpu.VMEM((seg, _CH), jnp.int32),
          pltpu.VMEM((_CH, d), jnp.float32),
          pltpu.VMEM_SHARED((_NPAD, d), jnp.float32),
          pltpu.SemaphoreType.DMA,
      ],
  )
  def k(table, gidx, sidx, out, vg, vs, rows, acc, sem):
    c = lax.axis_index("c")
    s = lax.axis_index("s")

    def zb(i, _):
      r = i // (d // 16)
      j = i % (d // 16)
      rows[r, pl.ds(j * 16, 16)] = jnp.zeros((16,), jnp.float32)
      return 0
    lax.fori_loop(0, 128 * (d // 16), zb, 0)

    def zacc(i, _):
      pltpu.sync_copy(rows, acc.at[pl.ds(s * _RPT + i * 128, 128)])
      return 0
    lax.fori_loop(0, _RPT // 128, zacc, 0)
    plsc.subcore_barrier()

    def chunk(j, _):
      pltpu.async_copy(table.at[c].at[vg.at[j]], rows, sem).wait()
      pltpu.sync_copy(rows, acc.at[vs.at[j]], add=True)
      return 0

    for half in range(2):
      pltpu.sync_copy(gidx.at[c, s, pl.ds(half * seg, seg)], vg)
      pltpu.sync_copy(sidx.at[c, s, pl.ds(half * seg, seg)], vs)
      lax.fori_loop(0, seg, chunk, 0)
    plsc.subcore_barrier()

    @pl.when(s < 15)
    def _():
      pltpu.sync_copy(acc.at[pl.ds(s * 624, 624)], out.at[c].at[pl.ds(s * 624, 624)])

    @pl.when(s == 15)
    def _():
      pltpu.sync_copy(acc.at[pl.ds(9360, 640)], out.at[c].at[pl.ds(9360, 640)])

  return k


_segsum_wide = _make_segsum(128, 160)
_segsum_split = _make_segsum(128, 80)


def _pad_idx(a, per, nch, padval):
  g = a.reshape(-1, per)
  pad = jnp.full((g.shape[0], nch * _CH - per), padval, jnp.int32)
  return jnp.concatenate([g, pad], axis=1).reshape(g.shape[0], nch, _CH)


def _mm0_body(x_ref, w_ref, b_ref, o_ref):
  r = jnp.dot(x_ref[...], w_ref[...].T, preferred_element_type=jnp.float32)
  r = r + b_ref[...]
  o_ref[0] = r[:, :128]
  o_ref[1] = r[:, 128:]


def _mm1_body(acc_ref, hl_ref, inv_ref, w_ref, b_ref, h_ref, o_ref):
  hcat = jnp.concatenate(
      [acc_ref[0] + hl_ref[0], acc_ref[1] + hl_ref[1]], axis=-1)
  t = jnp.tanh(hcat * inv_ref[...])
  h_ref[...] = t
  r = jnp.dot(t, w_ref[...].T, preferred_element_type=jnp.float32) + b_ref[...]
  o_ref[0] = r[:, :128]
  o_ref[1] = r[:, 128:]


_BM = 2000


def _mm0(x, w, b):
  return pl.pallas_call(
      _mm0_body,
      grid=(_N // _BM,),
      in_specs=[
          pl.BlockSpec((_BM, 128), lambda i: (i, 0)),
          pl.BlockSpec((256, 128), lambda i: (0, 0)),
          pl.BlockSpec((1, 256), lambda i: (0, 0)),
      ],
      out_specs=pl.BlockSpec((2, _BM, 128), lambda i: (0, i, 0)),
      out_shape=jax.ShapeDtypeStruct((2, _N, 128), jnp.float32),
  )(x, w, b.reshape(1, 256))


def _mm1(acc, hl, inv, w, b):
  return pl.pallas_call(
      _mm1_body,
      grid=(_N // _BM,),
      in_specs=[
          pl.BlockSpec((2, _BM, 128), lambda i: (0, i, 0)),
          pl.BlockSpec((2, _BM, 128), lambda i: (0, i, 0)),
          pl.BlockSpec((_BM, 1), lambda i: (i, 0)),
          pl.BlockSpec((256, 256), lambda i: (0, 0)),
          pl.BlockSpec((1, 256), lambda i: (0, 0)),
      ],
      out_specs=[
          pl.BlockSpec((_BM, 256), lambda i: (i, 0)),
          pl.BlockSpec((2, _BM, 128), lambda i: (0, i, 0)),
      ],
      out_shape=[
          jax.ShapeDtypeStruct((_N, 256), jnp.float32),
          jax.ShapeDtypeStruct((2, _N, 128), jnp.float32),
      ],
  )(acc, hl, inv, w, b.reshape(1, 256))


def kernel(x, edge_index, batch, W0, b0, W1, b1, W2, b2, W3, b3,
           Wc1, bc1, Wc2, bc2, Wd1, bd1, Wd2, bd2):
  src = edge_index[0]
  dst = edge_index[1]

  gw = jnp.broadcast_to(_pad_idx(src, 20000, 160, 0)[None], (2, 16, 160, _CH))
  sw = jnp.broadcast_to(_pad_idx(dst, 20000, 160, _N)[None], (2, 16, 160, _CH))
  gn = _pad_idx(src, 10000, 80, 0).reshape(2, 16, 80, _CH)
  sn = _pad_idx(dst, 10000, 80, _N).reshape(2, 16, 80, _CH)
  sn_deg = _pad_idx(src, 10000, 80, _N).reshape(2, 16, 80, _CH)

  ones_t = jnp.ones((2, _N, 128), jnp.float32)
  degp = _segsum_split(ones_t, gn, sn_deg)
  inv = (1.0 / (degp[0, :, 0] + degp[1, :, 0] + 1.0)).reshape(_N, 1)

  W3p = jnp.zeros((256, 256), jnp.float32).at[0].set(W3[0])
  b3p = jnp.zeros((256,), jnp.float32).at[0].set(b3[0])

  hl = _mm0(x, W0, b0)
  acc = _segsum_wide(hl, gw, sw)
  h1, hl = _mm1(acc, hl, inv, W1, b1)
  acc = _segsum_wide(hl, gw, sw)
  h2, hl = _mm1(acc, hl, inv, W2, b2)
  acc = _segsum_wide(hl, gw, sw)
  h3, hl = _mm1(acc, hl, inv, W3p, b3p)
  h4lin = hl[0, :, 0]
  t4 = jnp.broadcast_to(hl[0][None], (2, _N, 128))
  acc4 = _segsum_split(t4, gn, sn)
  h4 = jnp.tanh((acc4[0, :, 0] + acc4[1, :, 0] + h4lin) * inv[:, 0])

  xcat = jnp.concatenate([h1, h2, h3, h4[:, None]], axis=1)

  order = jnp.lexsort((-xcat[:, -1], batch))
  xs = xcat[order]
  bs = batch[order]
  counts = jnp.bincount(batch, length=_B)
  starts = jnp.cumsum(counts) - counts
  pos = jnp.arange(_N) - starts[bs]
  mask = pos < _K
  posc = jnp.where(mask, pos, _K - 1)
  vals = jnp.where(mask[:, None], xs, jnp.zeros_like(xs))
  pooled = jnp.zeros((_B, _K, 769), xcat.dtype).at[bs, posc].add(vals)

  c1 = jax.nn.relu(jnp.einsum('bkd,od->bok', pooled, Wc1[:, 0, :])
                   + bc1[None, :, None])
  c1p = jnp.max(c1.reshape(_B, 16, _K // 2, 2), axis=-1)
  Lout = _K // 2 - 4
  windows = jnp.stack([c1p[:, :, j:j + Lout] for j in range(5)], axis=-1)
  c2 = jax.nn.relu(jnp.einsum('bcij,ocj->boi', windows, Wc2)
                   + bc2[None, :, None])
  flat = c2.reshape(_B, -1)
  d1 = jax.nn.relu(flat @ Wd1.T + bd1)
  out = d1 @ Wd2.T + bd2
  return jax.nn.log_softmax(out, axis=-1)


# speedup vs baseline: 6.2663x; 1.2877x over previous
import functools

import jax
import jax.numpy as jnp
from jax import lax
from jax.experimental import pallas as pl
from jax.experimental.pallas import tpu as pltpu
from jax.experimental.pallas import tpu_sc as plsc

_N = 10000
_E = 320000
_B = 64
_K = 64
_NPAD = 10240
_RPT = _NPAD // 16
_VPT = _N // 16
_CH = 128


_SEG = 40


def _make_segsum(d, nch, const_ones=False):
  mesh = plsc.VectorSubcoreMesh(core_axis_name="c", subcore_axis_name="s")

  @functools.partial(
      pl.kernel,
      out_type=jax.ShapeDtypeStruct((2, _N, d), jnp.float32),
      mesh=mesh,
      scratch_types=[
          pltpu.VMEM((_SEG, _CH), jnp.int32),
          pltpu.VMEM((_SEG, _CH), jnp.int32),
          pltpu.VMEM((_CH, d), jnp.float32),
          pltpu.VMEM((_CH, d), jnp.float32),
          pltpu.VMEM_SHARED((_NPAD, d), jnp.float32),
          pltpu.SemaphoreType.DMA,
          pltpu.SemaphoreType.DMA,
          pltpu.SemaphoreType.DMA,
          pltpu.SemaphoreType.DMA,
      ],
  )
  def k(table, gidx, sidx, out, vg, vs, r0, r1, acc, g0, g1, s0, s1):
    c = lax.axis_index("c")
    s = lax.axis_index("s")
    bufs = ((r0, g0, s0), (r1, g1, s1))

    fill = 1.0 if const_ones else 0.0

    def zb(i, _):
      r = i // (d // 16)
      j = i % (d // 16)
      r0[r, pl.ds(j * 16, 16)] = jnp.full((16,), fill, jnp.float32)
      r1[r, pl.ds(j * 16, 16)] = jnp.zeros((16,), jnp.float32)
      return 0
    lax.fori_loop(0, 128 * (d // 16), zb, 0)

    def zacc(i, _):
      pltpu.sync_copy(r1, acc.at[pl.ds(s * _RPT + i * 128, 128)])
      return 0
    lax.fori_loop(0, _RPT // 128, zacc, 0)
    plsc.subcore_barrier()

    if const_ones:
      def chunk1(j, _):
        pltpu.sync_copy(r0, acc.at[vs.at[j]], add=True)
        return 0
      for part in range(nch // _SEG):
        pltpu.sync_copy(sidx.at[c, s, pl.ds(part * _SEG, _SEG)], vs)
        lax.fori_loop(0, _SEG, chunk1, 0)
    else:
      def gstart(j, buf, gsem):
        return pltpu.async_copy(table.at[c].at[vg.at[j]], buf, gsem)

      for part in range(nch // _SEG):
        pltpu.sync_copy(gidx.at[c, s, pl.ds(part * _SEG, _SEG)], vg)
        pltpu.sync_copy(sidx.at[c, s, pl.ds(part * _SEG, _SEG)], vs)
        gstart(0, r0, g0)
        gstart(1, r1, g1)

        def body(j2, _):
          for b, (buf, gsem, ssem) in enumerate(bufs):
            j = 2 * j2 + b
            pltpu.make_async_copy(table.at[c].at[vg.at[j]], buf, gsem).wait()
            cp = pltpu.async_copy(buf, acc.at[vs.at[j]], ssem, add=True)
            cp.wait()

            @pl.when(j + 2 < _SEG)
            def _():
              gstart(j + 2, buf, gsem)
          return 0
        lax.fori_loop(0, _SEG // 2, body, 0, unroll=False)
    plsc.subcore_barrier()

    @pl.when(s < 15)
    def _():
      pltpu.sync_copy(acc.at[pl.ds(s * 624, 624)], out.at[c].at[pl.ds(s * 624, 624)])

    @pl.when(s == 15)
    def _():
      pltpu.sync_copy(acc.at[pl.ds(9360, 640)], out.at[c].at[pl.ds(9360, 640)])

  return k


_segsum_wide = _make_segsum(128, 160)
_segsum_split = _make_segsum(128, 80)
_segsum_deg = _make_segsum(128, 80, const_ones=True)


def _pad_idx(a, per, nch, padval):
  g = a.reshape(-1, per)
  pad = jnp.full((g.shape[0], nch * _CH - per), padval, jnp.int32)
  return jnp.concatenate([g, pad], axis=1).reshape(g.shape[0], nch, _CH)


def _mm0_body(x_ref, w_ref, b_ref, o_ref):
  r = jnp.dot(x_ref[...], w_ref[...].T, preferred_element_type=jnp.float32)
  r = r + b_ref[...]
  o_ref[0] = r[:, :128]
  o_ref[1] = r[:, 128:]


def _mm1_body(acc_ref, hl_ref, inv_ref, w_ref, b_ref, h_ref, o_ref):
  hcat = jnp.concatenate(
      [acc_ref[0] + hl_ref[0], acc_ref[1] + hl_ref[1]], axis=-1)
  t = jnp.tanh(hcat * inv_ref[...])
  h_ref[...] = t
  r = jnp.dot(t, w_ref[...].T, preferred_element_type=jnp.float32) + b_ref[...]
  o_ref[0] = r[:, :128]
  o_ref[1] = r[:, 128:]


_BM = 2000


def _mm0(x, w, b):
  return pl.pallas_call(
      _mm0_body,
      grid=(_N // _BM,),
      in_specs=[
          pl.BlockSpec((_BM, 128), lambda i: (i, 0)),
          pl.BlockSpec((256, 128), lambda i: (0, 0)),
          pl.BlockSpec((1, 256), lambda i: (0, 0)),
      ],
      out_specs=pl.BlockSpec((2, _BM, 128), lambda i: (0, i, 0)),
      out_shape=jax.ShapeDtypeStruct((2, _N, 128), jnp.float32),
  )(x, w, b.reshape(1, 256))


def _mm1(acc, hl, inv, w, b):
  return pl.pallas_call(
      _mm1_body,
      grid=(_N // _BM,),
      in_specs=[
          pl.BlockSpec((2, _BM, 128), lambda i: (0, i, 0)),
          pl.BlockSpec((2, _BM, 128), lambda i: (0, i, 0)),
          pl.BlockSpec((_BM, 1), lambda i: (i, 0)),
          pl.BlockSpec((256, 256), lambda i: (0, 0)),
          pl.BlockSpec((1, 256), lambda i: (0, 0)),
      ],
      out_specs=[
          pl.BlockSpec((_BM, 256), lambda i: (i, 0)),
          pl.BlockSpec((2, _BM, 128), lambda i: (0, i, 0)),
      ],
      out_shape=[
          jax.ShapeDtypeStruct((_N, 256), jnp.float32),
          jax.ShapeDtypeStruct((2, _N, 128), jnp.float32),
      ],
  )(acc, hl, inv, w, b.reshape(1, 256))


def kernel(x, edge_index, batch, W0, b0, W1, b1, W2, b2, W3, b3,
           Wc1, bc1, Wc2, bc2, Wd1, bd1, Wd2, bd2):
  src = edge_index[0]
  dst = edge_index[1]

  gw = jnp.broadcast_to(_pad_idx(src, 20000, 160, 0)[None], (2, 16, 160, _CH))
  sw = jnp.broadcast_to(_pad_idx(dst, 20000, 160, _N)[None], (2, 16, 160, _CH))
  gn = _pad_idx(src, 10000, 80, 0).reshape(2, 16, 80, _CH)
  sn = _pad_idx(dst, 10000, 80, _N).reshape(2, 16, 80, _CH)
  sn_deg = _pad_idx(src, 10000, 80, _N).reshape(2, 16, 80, _CH)

  ones_t = jnp.ones((2, _N, 128), jnp.float32)
  degp = _segsum_deg(ones_t, gn, sn_deg)
  inv = (1.0 / (degp[0, :, 0] + degp[1, :, 0] + 1.0)).reshape(_N, 1)

  W3p = jnp.zeros((256, 256), jnp.float32).at[0].set(W3[0])
  b3p = jnp.zeros((256,), jnp.float32).at[0].set(b3[0])

  hl = _mm0(x, W0, b0)
  acc = _segsum_wide(hl, gw, sw)
  h1, hl = _mm1(acc, hl, inv, W1, b1)
  acc = _segsum_wide(hl, gw, sw)
  h2, hl = _mm1(acc, hl, inv, W2, b2)
  acc = _segsum_wide(hl, gw, sw)
  h3, hl = _mm1(acc, hl, inv, W3p, b3p)
  h4lin = hl[0, :, 0]
  t4 = jnp.broadcast_to(hl[0][None], (2, _N, 128))
  acc4 = _segsum_split(t4, gn, sn)
  h4 = jnp.tanh((acc4[0, :, 0] + acc4[1, :, 0] + h4lin) * inv[:, 0])

  xcat = jnp.concatenate([h1, h2, h3, h4[:, None]], axis=1)

  order = jnp.lexsort((-xcat[:, -1], batch))
  xs = xcat[order]
  bs = batch[order]
  counts = jnp.bincount(batch, length=_B)
  starts = jnp.cumsum(counts) - counts
  pos = jnp.arange(_N) - starts[bs]
  mask = pos < _K
  posc = jnp.where(mask, pos, _K - 1)
  vals = jnp.where(mask[:, None], xs, jnp.zeros_like(xs))
  pooled = jnp.zeros((_B, _K, 769), xcat.dtype).at[bs, posc].add(vals)

  c1 = jax.nn.relu(jnp.einsum('bkd,od->bok', pooled, Wc1[:, 0, :])
                   + bc1[None, :, None])
  c1p = jnp.max(c1.reshape(_B, 16, _K // 2, 2), axis=-1)
  Lout = _K // 2 - 4
  windows = jnp.stack([c1p[:, :, j:j + Lout] for j in range(5)], axis=-1)
  c2 = jax.nn.relu(jnp.einsum('bcij,ocj->boi', windows, Wc2)
                   + bc2[None, :, None])
  flat = c2.reshape(_B, -1)
  d1 = jax.nn.relu(flat @ Wd1.T + bd1)
  out = d1 @ Wd2.T + bd2
  return jax.nn.log_softmax(out, axis=-1)
